# baseline (speedup 1.0000x reference)
import functools
import math

import jax
import jax.numpy as jnp
from jax import lax
from jax.experimental import pallas as pl
from jax.experimental.pallas import tpu as pltpu
from jax.experimental.pallas import tpu_sc as plsc

N_NODES = 10000
DIM = 128
ALPHA_ITERS = 20
BLOCKS = 2

_ROWS = 632

_NC = 2
_NS = 16
_NW = _NC * _NS
_L = 16
_CHUNK = 128


def _sc_mesh():
    return plsc.VectorSubcoreMesh(
        core_axis_name="c", subcore_axis_name="s", num_cores=_NC, num_subcores=_NS)


def _sc_seg_pass(table, gidx, sidx, zeros, n, acc_rows, k_chunks):
    z_per_tile = acc_rows // _NS
    kp = k_chunks // 2

    def body(table_h, gidx_h, sidx_h, zeros_h, out_h,
             gidx_v, sidx_v, rb0, rb1, acc_sh, semA, semB, semC, semD):
        c = lax.axis_index("c")
        s = lax.axis_index("s")
        wid = c * _NS + s
        pltpu.sync_copy(zeros_h.at[pl.ds(s * z_per_tile, z_per_tile)],
                        acc_sh.at[pl.ds(s * z_per_tile, z_per_tile)])
        plsc.subcore_barrier()

        for p in range(2):
            pltpu.sync_copy(gidx_h.at[wid, pl.ds(p * kp, kp)], gidx_v)
            pltpu.sync_copy(sidx_h.at[wid, pl.ds(p * kp, kp)], sidx_v)

            def pair(i, carry):
                k0 = 2 * i
                g0 = pltpu.async_copy(table_h.at[gidx_v.at[k0]], rb0, semA)
                g1 = pltpu.async_copy(table_h.at[gidx_v.at[k0 + 1]], rb1, semB)
                g0.wait()
                s0 = pltpu.async_copy(rb0, acc_sh.at[sidx_v.at[k0]],
                                      semC, add=True)
                g1.wait()
                s1 = pltpu.async_copy(rb1, acc_sh.at[sidx_v.at[k0 + 1]],
                                      semD, add=True)
                s0.wait()
                s1.wait()
                return carry

            lax.fori_loop(0, kp // 2, pair, 0, unroll=False)

        plsc.subcore_barrier()
        pltpu.sync_copy(
            acc_sh.at[pl.ds(s * z_per_tile, z_per_tile)],
            out_h.at[pl.ds(c * acc_rows + s * z_per_tile, z_per_tile)])

    return pl.kernel(
        body,
        out_type=jax.ShapeDtypeStruct((2 * acc_rows, DIM), jnp.float32),
        mesh=_sc_mesh(),
        scratch_types=[
            pltpu.VMEM((k_chunks // 2, _CHUNK), jnp.int32),
            pltpu.VMEM((k_chunks // 2, _CHUNK), jnp.int32),
            pltpu.VMEM((_CHUNK, DIM), jnp.float32),
            pltpu.VMEM((_CHUNK, DIM), jnp.float32),
            pltpu.VMEM_SHARED((acc_rows, DIM), jnp.float32),
            pltpu.SemaphoreType.DMA,
            pltpu.SemaphoreType.DMA,
            pltpu.SemaphoreType.DMA,
            pltpu.SemaphoreType.DMA,
        ],
    )(table, gidx, sidx, zeros)


def _row_spec():
    return pl.BlockSpec((_ROWS, DIM), lambda i: (i, 0))


def _col_spec():
    return pl.BlockSpec((_ROWS, 1), lambda i: (i, 0))


def _stage_a_body(x_ref, wl_ref, bl_ref, w1_ref, w2_ref, u_ref, v_ref, d_ref,
                  out0_ref, a_ref, b_ref, dh_ref, y_ref, yt_ref):
    x = x_ref[...]
    out0_ref[...] = jnp.dot(x, wl_ref[...], preferred_element_type=jnp.float32) + bl_ref[...]
    h1 = jnp.dot(x, w1_ref[...], preferred_element_type=jnp.float32)
    y = jnp.dot(x, w2_ref[...], preferred_element_type=jnp.float32)
    a_ref[...] = u_ref[...] * h1
    b_ref[...] = v_ref[...] * h1
    dh_ref[...] = d_ref[...] * h1
    y_ref[...] = y
    yt_ref[...] = d_ref[...] * y


def _stage_a(x, W_lin, b_lin, W1, W2, u, v, dinv):
    n = x.shape[0]
    grid = (n // _ROWS,)
    full_spec = pl.BlockSpec((DIM, DIM), lambda i: (0, 0))
    bias_spec = pl.BlockSpec((1, DIM), lambda i: (0, 0))
    return pl.pallas_call(
        _stage_a_body,
        grid=grid,
        in_specs=[_row_spec(), full_spec, bias_spec, full_spec, full_spec,
                  _col_spec(), _col_spec(), _col_spec()],
        out_specs=[_row_spec()] * 6,
        out_shape=[jax.ShapeDtypeStruct((n, DIM), jnp.float32)] * 6,
    )(x, W_lin, b_lin.reshape(1, DIM), W1, W2, u, v, dinv)


def _stage_b_body(t1a_ref, t1b_ref, y_ref, t2a_ref, t2b_ref, yt_ref, d_ref,
                  pys_ref, pty_ref):
    d = d_ref[...]
    pys_ref[...] = d * d * (t1a_ref[...] + t1b_ref[...] + y_ref[...])
    pty_ref[...] = t2a_ref[...] + t2b_ref[...] + yt_ref[...]


def _stage_b(t1, y, t2, yt, dinv):
    n = y.shape[0]
    grid = (n // _ROWS,)
    lo = pl.BlockSpec((_ROWS, DIM), lambda i: (i, 0))
    hi = pl.BlockSpec((_ROWS, DIM), lambda i: (i + n // _ROWS, 0))
    return pl.pallas_call(
        _stage_b_body,
        grid=grid,
        in_specs=[lo, hi, _row_spec(), lo, hi, _row_spec(), _col_spec()],
        out_specs=[_row_spec()] * 2,
        out_shape=[jax.ShapeDtypeStruct((n, DIM), jnp.float32)] * 2,
    )(t1, t1, y, t2, t2, yt, dinv)


def _stage_c_body(out0_ref, s1a_ref, s1b_ref, s2a_ref, s2b_ref, dh_ref,
                  t3a_ref, t3b_ref, t4a_ref, t4b_ref, pys_ref, pty_ref,
                  u_ref, v_ref, d_ref, b1_ref, b2_ref, x_ref):
    out1 = (v_ref[...] * (s1a_ref[...] + s1b_ref[...])
            + u_ref[...] * (s2a_ref[...] + s2b_ref[...])
            + dh_ref[...] + b1_ref[...])
    l_in = t3a_ref[...] + t3b_ref[...] + pys_ref[...]
    l_out = d_ref[...] * (t4a_ref[...] + t4b_ref[...] + pty_ref[...])
    x_ref[...] = out0_ref[...] + out1 + 0.5 * (l_in + l_out) + b2_ref[...]


def _stage_c(out0, s1, s2, dh, t3, t4, pys, pty, u, v, dinv, b1, b2):
    n = out0.shape[0]
    grid = (n // _ROWS,)
    lo = pl.BlockSpec((_ROWS, DIM), lambda i: (i, 0))
    hi = pl.BlockSpec((_ROWS, DIM), lambda i: (i + n // _ROWS, 0))
    bias_spec = pl.BlockSpec((1, DIM), lambda i: (0, 0))
    return pl.pallas_call(
        _stage_c_body,
        grid=grid,
        in_specs=[_row_spec(), lo, hi, lo, hi, _row_spec(), lo, hi, lo, hi,
                  _row_spec(), _row_spec(), _col_spec(), _col_spec(),
                  _col_spec(), bias_spec, bias_spec],
        out_specs=_row_spec(),
        out_shape=jax.ShapeDtypeStruct((n, DIM), jnp.float32),
    )(out0, s1, s1, s2, s2, dh, t3, t3, t4, t4, pys, pty, u, v, dinv,
      b1.reshape(1, DIM), b2.reshape(1, DIM))


def _scalar_prep_body(pi_ref, d_ref, u_ref, v_ref):
    pis = jnp.sqrt(jnp.clip(pi_ref[...], 1e-12, None))
    u_ref[...] = 0.5 * d_ref[...] * pis
    v_ref[...] = 1.0 / pis


def _scalar_prep(pi_p, dinv_p):
    shp = pi_p.shape
    return pl.pallas_call(
        _scalar_prep_body,
        out_shape=[jax.ShapeDtypeStruct(shp, jnp.float32)] * 2,
    )(pi_p, dinv_p)


def _sc_power_iter(srcf, dstf, ca_in, cb_in, np_rows):
    rpt = np_rows // _NS
    nv = rpt // _L
    n_ch = srcf.shape[1]
    fpc = _CHUNK // _L

    def body(srcf_h, dstf_h, ca_h, cb_h, pi_h, dinv_h,
             src2, dst2, q_tab, vb, ones_b, zbuf, slc, dinv_l, pi_l, qs,
             ca, cb, sem0, sem1, acc_sh, q_sh):
        c = lax.axis_index("c")
        s = lax.axis_index("s")

        @pl.when(c == 0)
        def _run():
            base = s * rpt
            pltpu.sync_copy(srcf_h.at[s], src2)
            pltpu.sync_copy(dstf_h.at[s], dst2)
            pltpu.sync_copy(ca_h, ca)
            pltpu.sync_copy(cb_h, cb)
            zeros16 = jnp.zeros((_L,), jnp.float32)
            ones16 = jnp.ones((_L,), jnp.float32)
            for t in range(fpc):
                ones_b[pl.ds(t * _L, _L)] = ones16

            def zero16(j, _):
                zbuf[pl.ds(j * _L, _L)] = zeros16
                return _

            lax.fori_loop(0, nv, zero16, 0)
            pltpu.sync_copy(zbuf, acc_sh.at[pl.ds(base, rpt)])
            plsc.subcore_barrier()

            def hpair(i, _):
                d0 = pltpu.async_copy(ones_b, acc_sh.at[src2.at[2 * i]],
                                      sem0, add=True)
                d1 = pltpu.async_copy(ones_b, acc_sh.at[src2.at[2 * i + 1]],
                                      sem1, add=True)
                d0.wait()
                d1.wait()
                return _

            lax.fori_loop(0, n_ch // 2, hpair, 0)
            plsc.subcore_barrier()
            pltpu.sync_copy(acc_sh.at[pl.ds(base, rpt)], slc)
            inv_n = 1.0 / float(N_NODES)

            def dstep(j, _):
                di = 1.0 / (slc[pl.ds(j * _L, _L)] + 1.0)
                dinv_l[pl.ds(j * _L, _L)] = di
                qs[pl.ds(j * _L, _L)] = di * inv_n
                return _

            lax.fori_loop(0, nv, dstep, 0)
            pltpu.sync_copy(qs, q_sh.at[pl.ds(base, rpt)])
            pltpu.sync_copy(zbuf, acc_sh.at[pl.ds(base, rpt)])
            plsc.subcore_barrier()
            pltpu.sync_copy(q_sh, q_tab)

            def gath(k, b, sem):
                return pltpu.async_copy(q_sh.at[src2.at[k]], vb.at[b], sem)

            def one_iter(it, _):
                g0 = gath(0, 0, sem0)
                g0.wait()

                def pair(i, _c):
                    k0 = 2 * i
                    d0 = pltpu.async_copy(vb.at[0], acc_sh.at[dst2.at[k0]],
                                          sem0, add=True)
                    g1 = gath(k0 + 1, 1, sem1)
                    g1.wait()
                    d1 = pltpu.async_copy(vb.at[1], acc_sh.at[dst2.at[k0 + 1]],
                                          sem1, add=True)
                    d0.wait()

                    @pl.when(k0 + 2 < n_ch)
                    def _():
                        gath(k0 + 2, 0, sem0).wait()

                    d1.wait()
                    return _c

                lax.fori_loop(0, n_ch // 2, pair, 0)
                plsc.subcore_barrier()
                pltpu.sync_copy(acc_sh.at[pl.ds(base, rpt)], slc)
                A = ca[...]
                B = cb[...]

                def ustep(j, _c):
                    acc16 = slc[pl.ds(j * _L, _L)] + q_tab[pl.ds(base + j * _L, _L)]
                    pi16 = A * acc16 + B
                    pi_l[pl.ds(j * _L, _L)] = pi16
                    qs[pl.ds(j * _L, _L)] = dinv_l[pl.ds(j * _L, _L)] * pi16
                    return _c

                lax.fori_loop(0, nv, ustep, 0)
                pltpu.sync_copy(qs, q_sh.at[pl.ds(base, rpt)])
                pltpu.sync_copy(zbuf, acc_sh.at[pl.ds(base, rpt)])
                plsc.subcore_barrier()
                pltpu.sync_copy(q_sh, q_tab)
                return _

            lax.fori_loop(0, ALPHA_ITERS, one_iter, 0)
            pltpu.sync_copy(pi_l, pi_h.at[pl.ds(base, rpt)])
            pltpu.sync_copy(dinv_l, dinv_h.at[pl.ds(base, rpt)])

    return pl.kernel(
        body,
        out_type=(jax.ShapeDtypeStruct((np_rows,), jnp.float32),
                  jax.ShapeDtypeStruct((np_rows,), jnp.float32)),
        mesh=_sc_mesh(),
        scratch_types=[
            pltpu.VMEM((n_ch, _CHUNK), jnp.int32),
            pltpu.VMEM((n_ch, _CHUNK), jnp.int32),
            pltpu.VMEM((np_rows,), jnp.float32),
            pltpu.VMEM((2, _CHUNK), jnp.float32),
            pltpu.VMEM((_CHUNK,), jnp.float32),
            pltpu.VMEM((np_rows // _NS,), jnp.float32),
            pltpu.VMEM((np_rows // _NS,), jnp.float32),
            pltpu.VMEM((np_rows // _NS,), jnp.float32),
            pltpu.VMEM((np_rows // _NS,), jnp.float32),
            pltpu.VMEM((np_rows // _NS,), jnp.float32),
            pltpu.VMEM((_L,), jnp.float32),
            pltpu.VMEM((_L,), jnp.float32),
            pltpu.SemaphoreType.DMA,
            pltpu.SemaphoreType.DMA,
            pltpu.VMEM_SHARED((np_rows,), jnp.float32),
            pltpu.VMEM_SHARED((np_rows,), jnp.float32),
        ],
    )(srcf, dstf, ca_in, cb_in)


def kernel(x, edge_index, alpha, W_lin, b_lin, W1, b1, W2, b2):
    n = x.shape[0]
    src = edge_index[0]
    dst = edge_index[1]

    np_rows = -(-(n + 1) // (_NS * _L * 8)) * (_NS * _L * 8)
    ca_in = jnp.full((_L,), 1.0, jnp.float32) - alpha
    cb_in = jnp.full((_L,), 1.0 / n, jnp.float32) * alpha
    ept = src.shape[0] // _NS
    ept_pad = -(-ept // (2 * _CHUNK)) * (2 * _CHUNK)
    srcp = jnp.pad(src.reshape(_NS, ept), ((0, 0), (0, ept_pad - ept)),
                   constant_values=n).reshape(_NS, -1, _CHUNK)
    dstp = jnp.pad(dst.reshape(_NS, ept), ((0, 0), (0, ept_pad - ept)),
                   constant_values=n).reshape(_NS, -1, _CHUNK)
    pi_p, dinv_p = _sc_power_iter(srcp, dstp, ca_in, cb_in, np_rows)
    u_p, v_p = _scalar_prep(pi_p, dinv_p)

    e = src.shape[0]
    e_pad = -(-e // (4 * _NW * _CHUNK)) * (4 * _NW * _CHUNK)
    k_chunks = e_pad // (_NW * _CHUNK)
    blk = math.lcm(_NS * 8, _ROWS)
    acc_rows = -(-(n + 1) // blk) * blk
    srcg = jnp.pad(src, (0, e_pad - e)).reshape(_NW, k_chunks, _CHUNK)
    dstg = jnp.pad(dst, (0, e_pad - e)).reshape(_NW, k_chunks, _CHUNK)
    srcs = jnp.pad(src, (0, e_pad - e), constant_values=n).reshape(_NW, k_chunks, _CHUNK)
    dsts = jnp.pad(dst, (0, e_pad - e), constant_values=n).reshape(_NW, k_chunks, _CHUNK)
    zeros = jnp.zeros((acc_rows, DIM), jnp.float32)

    u = u_p.reshape(-1)[:acc_rows].reshape(acc_rows, 1)
    v = v_p.reshape(-1)[:acc_rows].reshape(acc_rows, 1)
    dinv = dinv_p.reshape(-1)[:acc_rows].reshape(acc_rows, 1)
    xp = jnp.pad(x, ((0, acc_rows - n), (0, 0)))

    def seg(table, gi, si):
        return _sc_seg_pass(table, gi, si, zeros, n, acc_rows, k_chunks)

    for _ in range(BLOCKS):
        out0, a_tab, b_tab, dh, y, yt = _stage_a(xp, W_lin, b_lin, W1, W2,
                                                 u, v, dinv)
        s1 = seg(a_tab, srcg, dsts)
        s2 = seg(b_tab, dstg, srcs)
        t1 = seg(y, dstg, srcs)
        t2 = seg(yt, srcg, dsts)

        pys, pty = _stage_b(t1, y, t2, yt, dinv)

        t3 = seg(pys, srcg, dsts)
        t4 = seg(pty, dstg, srcs)

        xp = _stage_c(out0, s1, s2, dh, t3, t4, pys, pty, u, v, dinv, b1, b2)
    return xp[:n]


# speedup vs baseline: 1.0159x; 1.0159x over previous
import functools
import math

import jax
import jax.numpy as jnp
from jax import lax
from jax.experimental import pallas as pl
from jax.experimental.pallas import tpu as pltpu
from jax.experimental.pallas import tpu_sc as plsc

N_NODES = 10000
DIM = 128
ALPHA_ITERS = 20
BLOCKS = 2

_ROWS = 632

_NC = 2
_NS = 16
_NW = _NC * _NS
_L = 16
_CHUNK = 128


def _sc_mesh():
    return plsc.VectorSubcoreMesh(
        core_axis_name="c", subcore_axis_name="s", num_cores=_NC, num_subcores=_NS)


def _sc_seg_pass(table, gidx, sidx, zeros, n, acc_rows, k_chunks):
    z_per_tile = acc_rows // _NS
    kp = k_chunks // 2

    def body(table_h, gidx_h, sidx_h, zeros_h, out_h,
             gidx_v, sidx_v, rb0, rb1, acc_sh, semA, semB, semC, semD):
        c = lax.axis_index("c")
        s = lax.axis_index("s")
        wid = c * _NS + s
        pltpu.sync_copy(zeros_h.at[pl.ds(s * z_per_tile, z_per_tile)],
                        acc_sh.at[pl.ds(s * z_per_tile, z_per_tile)])
        plsc.subcore_barrier()

        for p in range(2):
            pltpu.sync_copy(gidx_h.at[wid, pl.ds(p * kp, kp)], gidx_v)
            pltpu.sync_copy(sidx_h.at[wid, pl.ds(p * kp, kp)], sidx_v)

            def pair(i, carry):
                k0 = 2 * i
                g0 = pltpu.async_copy(table_h.at[gidx_v.at[k0]], rb0, semA)
                g0.wait()
                s0 = pltpu.async_copy(rb0, acc_sh.at[sidx_v.at[k0]],
                                      semC, add=True)
                g1 = pltpu.async_copy(table_h.at[gidx_v.at[k0 + 1]], rb1, semB)
                g1.wait()
                s0.wait()
                s1 = pltpu.async_copy(rb1, acc_sh.at[sidx_v.at[k0 + 1]],
                                      semD, add=True)
                s1.wait()
                return carry

            lax.fori_loop(0, kp // 2, pair, 0, unroll=False)

        plsc.subcore_barrier()
        pltpu.sync_copy(
            acc_sh.at[pl.ds(s * z_per_tile, z_per_tile)],
            out_h.at[pl.ds(c * acc_rows + s * z_per_tile, z_per_tile)])

    return pl.kernel(
        body,
        out_type=jax.ShapeDtypeStruct((2 * acc_rows, DIM), jnp.float32),
        mesh=_sc_mesh(),
        scratch_types=[
            pltpu.VMEM((k_chunks // 2, _CHUNK), jnp.int32),
            pltpu.VMEM((k_chunks // 2, _CHUNK), jnp.int32),
            pltpu.VMEM((_CHUNK, DIM), jnp.float32),
            pltpu.VMEM((_CHUNK, DIM), jnp.float32),
            pltpu.VMEM_SHARED((acc_rows, DIM), jnp.float32),
            pltpu.SemaphoreType.DMA,
            pltpu.SemaphoreType.DMA,
            pltpu.SemaphoreType.DMA,
            pltpu.SemaphoreType.DMA,
        ],
    )(table, gidx, sidx, zeros)


def _row_spec():
    return pl.BlockSpec((_ROWS, DIM), lambda i: (i, 0))


def _col_spec():
    return pl.BlockSpec((_ROWS, 1), lambda i: (i, 0))


def _stage_a_body(x_ref, wl_ref, bl_ref, w1_ref, w2_ref, u_ref, v_ref, d_ref,
                  out0_ref, a_ref, b_ref, dh_ref, y_ref, yt_ref):
    x = x_ref[...]
    out0_ref[...] = jnp.dot(x, wl_ref[...], preferred_element_type=jnp.float32) + bl_ref[...]
    h1 = jnp.dot(x, w1_ref[...], preferred_element_type=jnp.float32)
    y = jnp.dot(x, w2_ref[...], preferred_element_type=jnp.float32)
    a_ref[...] = u_ref[...] * h1
    b_ref[...] = v_ref[...] * h1
    dh_ref[...] = d_ref[...] * h1
    y_ref[...] = y
    yt_ref[...] = d_ref[...] * y


def _stage_a(x, W_lin, b_lin, W1, W2, u, v, dinv):
    n = x.shape[0]
    grid = (n // _ROWS,)
    full_spec = pl.BlockSpec((DIM, DIM), lambda i: (0, 0))
    bias_spec = pl.BlockSpec((1, DIM), lambda i: (0, 0))
    return pl.pallas_call(
        _stage_a_body,
        grid=grid,
        in_specs=[_row_spec(), full_spec, bias_spec, full_spec, full_spec,
                  _col_spec(), _col_spec(), _col_spec()],
        out_specs=[_row_spec()] * 6,
        out_shape=[jax.ShapeDtypeStruct((n, DIM), jnp.float32)] * 6,
    )(x, W_lin, b_lin.reshape(1, DIM), W1, W2, u, v, dinv)


def _stage_b_body(t1a_ref, t1b_ref, y_ref, t2a_ref, t2b_ref, yt_ref, d_ref,
                  pys_ref, pty_ref):
    d = d_ref[...]
    pys_ref[...] = d * d * (t1a_ref[...] + t1b_ref[...] + y_ref[...])
    pty_ref[...] = t2a_ref[...] + t2b_ref[...] + yt_ref[...]


def _stage_b(t1, y, t2, yt, dinv):
    n = y.shape[0]
    grid = (n // _ROWS,)
    lo = pl.BlockSpec((_ROWS, DIM), lambda i: (i, 0))
    hi = pl.BlockSpec((_ROWS, DIM), lambda i: (i + n // _ROWS, 0))
    return pl.pallas_call(
        _stage_b_body,
        grid=grid,
        in_specs=[lo, hi, _row_spec(), lo, hi, _row_spec(), _col_spec()],
        out_specs=[_row_spec()] * 2,
        out_shape=[jax.ShapeDtypeStruct((n, DIM), jnp.float32)] * 2,
    )(t1, t1, y, t2, t2, yt, dinv)


def _stage_c_body(out0_ref, s1a_ref, s1b_ref, s2a_ref, s2b_ref, dh_ref,
                  t3a_ref, t3b_ref, t4a_ref, t4b_ref, pys_ref, pty_ref,
                  u_ref, v_ref, d_ref, b1_ref, b2_ref, x_ref):
    out1 = (v_ref[...] * (s1a_ref[...] + s1b_ref[...])
            + u_ref[...] * (s2a_ref[...] + s2b_ref[...])
            + dh_ref[...] + b1_ref[...])
    l_in = t3a_ref[...] + t3b_ref[...] + pys_ref[...]
    l_out = d_ref[...] * (t4a_ref[...] + t4b_ref[...] + pty_ref[...])
    x_ref[...] = out0_ref[...] + out1 + 0.5 * (l_in + l_out) + b2_ref[...]


def _stage_c(out0, s1, s2, dh, t3, t4, pys, pty, u, v, dinv, b1, b2):
    n = out0.shape[0]
    grid = (n // _ROWS,)
    lo = pl.BlockSpec((_ROWS, DIM), lambda i: (i, 0))
    hi = pl.BlockSpec((_ROWS, DIM), lambda i: (i + n // _ROWS, 0))
    bias_spec = pl.BlockSpec((1, DIM), lambda i: (0, 0))
    return pl.pallas_call(
        _stage_c_body,
        grid=grid,
        in_specs=[_row_spec(), lo, hi, lo, hi, _row_spec(), lo, hi, lo, hi,
                  _row_spec(), _row_spec(), _col_spec(), _col_spec(),
                  _col_spec(), bias_spec, bias_spec],
        out_specs=_row_spec(),
        out_shape=jax.ShapeDtypeStruct((n, DIM), jnp.float32),
    )(out0, s1, s1, s2, s2, dh, t3, t3, t4, t4, pys, pty, u, v, dinv,
      b1.reshape(1, DIM), b2.reshape(1, DIM))


def _scalar_prep_body(pi_ref, d_ref, u_ref, v_ref):
    pis = jnp.sqrt(jnp.clip(pi_ref[...], 1e-12, None))
    u_ref[...] = 0.5 * d_ref[...] * pis
    v_ref[...] = 1.0 / pis


def _scalar_prep(pi_p, dinv_p):
    shp = pi_p.shape
    return pl.pallas_call(
        _scalar_prep_body,
        out_shape=[jax.ShapeDtypeStruct(shp, jnp.float32)] * 2,
    )(pi_p, dinv_p)


def _sc_power_iter(srcf, dstf, ca_in, cb_in, np_rows):
    rpt = np_rows // _NS
    nv = rpt // _L
    n_ch = srcf.shape[1]
    fpc = _CHUNK // _L

    def body(srcf_h, dstf_h, ca_h, cb_h, pi_h, dinv_h,
             src2, dst2, q_tab, vb, ones_b, zbuf, slc, dinv_l, pi_l, qs,
             ca, cb, sem0, sem1, acc_sh, q_sh):
        c = lax.axis_index("c")
        s = lax.axis_index("s")

        @pl.when(c == 0)
        def _run():
            base = s * rpt
            pltpu.sync_copy(srcf_h.at[s], src2)
            pltpu.sync_copy(dstf_h.at[s], dst2)
            pltpu.sync_copy(ca_h, ca)
            pltpu.sync_copy(cb_h, cb)
            zeros16 = jnp.zeros((_L,), jnp.float32)
            ones16 = jnp.ones((_L,), jnp.float32)
            for t in range(fpc):
                ones_b[pl.ds(t * _L, _L)] = ones16

            def zero16(j, _):
                zbuf[pl.ds(j * _L, _L)] = zeros16
                return _

            lax.fori_loop(0, nv, zero16, 0)
            pltpu.sync_copy(zbuf, acc_sh.at[pl.ds(base, rpt)])
            plsc.subcore_barrier()

            def hpair(i, _):
                d0 = pltpu.async_copy(ones_b, acc_sh.at[src2.at[2 * i]],
                                      sem0, add=True)
                d1 = pltpu.async_copy(ones_b, acc_sh.at[src2.at[2 * i + 1]],
                                      sem1, add=True)
                d0.wait()
                d1.wait()
                return _

            lax.fori_loop(0, n_ch // 2, hpair, 0)
            plsc.subcore_barrier()
            pltpu.sync_copy(acc_sh.at[pl.ds(base, rpt)], slc)
            inv_n = 1.0 / float(N_NODES)

            def dstep(j, _):
                di = 1.0 / (slc[pl.ds(j * _L, _L)] + 1.0)
                dinv_l[pl.ds(j * _L, _L)] = di
                qs[pl.ds(j * _L, _L)] = di * inv_n
                return _

            lax.fori_loop(0, nv, dstep, 0)
            pltpu.sync_copy(qs, q_sh.at[pl.ds(base, rpt)])
            pltpu.sync_copy(zbuf, acc_sh.at[pl.ds(base, rpt)])
            plsc.subcore_barrier()
            pltpu.sync_copy(q_sh, q_tab)

            def gath(k, b, sem):
                return pltpu.async_copy(q_sh.at[src2.at[k]], vb.at[b], sem)

            def one_iter(it, _):
                g0 = gath(0, 0, sem0)
                g0.wait()

                def pair(i, _c):
                    k0 = 2 * i
                    d0 = pltpu.async_copy(vb.at[0], acc_sh.at[dst2.at[k0]],
                                          sem0, add=True)
                    g1 = gath(k0 + 1, 1, sem1)
                    g1.wait()
                    d1 = pltpu.async_copy(vb.at[1], acc_sh.at[dst2.at[k0 + 1]],
                                          sem1, add=True)
                    d0.wait()

                    @pl.when(k0 + 2 < n_ch)
                    def _():
                        gath(k0 + 2, 0, sem0).wait()

                    d1.wait()
                    return _c

                lax.fori_loop(0, n_ch // 2, pair, 0)
                plsc.subcore_barrier()
                pltpu.sync_copy(acc_sh.at[pl.ds(base, rpt)], slc)
                A = ca[...]
                B = cb[...]

                def ustep(j, _c):
                    acc16 = slc[pl.ds(j * _L, _L)] + q_tab[pl.ds(base + j * _L, _L)]
                    pi16 = A * acc16 + B
                    pi_l[pl.ds(j * _L, _L)] = pi16
                    qs[pl.ds(j * _L, _L)] = dinv_l[pl.ds(j * _L, _L)] * pi16
                    return _c

                lax.fori_loop(0, nv, ustep, 0)
                pltpu.sync_copy(qs, q_sh.at[pl.ds(base, rpt)])
                pltpu.sync_copy(zbuf, acc_sh.at[pl.ds(base, rpt)])
                plsc.subcore_barrier()
                pltpu.sync_copy(q_sh, q_tab)
                return _

            lax.fori_loop(0, ALPHA_ITERS, one_iter, 0)
            pltpu.sync_copy(pi_l, pi_h.at[pl.ds(base, rpt)])
            pltpu.sync_copy(dinv_l, dinv_h.at[pl.ds(base, rpt)])

    return pl.kernel(
        body,
        out_type=(jax.ShapeDtypeStruct((np_rows,), jnp.float32),
                  jax.ShapeDtypeStruct((np_rows,), jnp.float32)),
        mesh=_sc_mesh(),
        scratch_types=[
            pltpu.VMEM((n_ch, _CHUNK), jnp.int32),
            pltpu.VMEM((n_ch, _CHUNK), jnp.int32),
            pltpu.VMEM((np_rows,), jnp.float32),
            pltpu.VMEM((2, _CHUNK), jnp.float32),
            pltpu.VMEM((_CHUNK,), jnp.float32),
            pltpu.VMEM((np_rows // _NS,), jnp.float32),
            pltpu.VMEM((np_rows // _NS,), jnp.float32),
            pltpu.VMEM((np_rows // _NS,), jnp.float32),
            pltpu.VMEM((np_rows // _NS,), jnp.float32),
            pltpu.VMEM((np_rows // _NS,), jnp.float32),
            pltpu.VMEM((_L,), jnp.float32),
            pltpu.VMEM((_L,), jnp.float32),
            pltpu.SemaphoreType.DMA,
            pltpu.SemaphoreType.DMA,
            pltpu.VMEM_SHARED((np_rows,), jnp.float32),
            pltpu.VMEM_SHARED((np_rows,), jnp.float32),
        ],
    )(srcf, dstf, ca_in, cb_in)


def kernel(x, edge_index, alpha, W_lin, b_lin, W1, b1, W2, b2):
    n = x.shape[0]
    src = edge_index[0]
    dst = edge_index[1]

    np_rows = -(-(n + 1) // (_NS * _L * 8)) * (_NS * _L * 8)
    ca_in = jnp.full((_L,), 1.0, jnp.float32) - alpha
    cb_in = jnp.full((_L,), 1.0 / n, jnp.float32) * alpha
    ept = src.shape[0] // _NS
    ept_pad = -(-ept // (2 * _CHUNK)) * (2 * _CHUNK)
    srcp = jnp.pad(src.reshape(_NS, ept), ((0, 0), (0, ept_pad - ept)),
                   constant_values=n).reshape(_NS, -1, _CHUNK)
    dstp = jnp.pad(dst.reshape(_NS, ept), ((0, 0), (0, ept_pad - ept)),
                   constant_values=n).reshape(_NS, -1, _CHUNK)
    pi_p, dinv_p = _sc_power_iter(srcp, dstp, ca_in, cb_in, np_rows)
    u_p, v_p = _scalar_prep(pi_p, dinv_p)

    e = src.shape[0]
    e_pad = -(-e // (4 * _NW * _CHUNK)) * (4 * _NW * _CHUNK)
    k_chunks = e_pad // (_NW * _CHUNK)
    blk = math.lcm(_NS * 8, _ROWS)
    acc_rows = -(-(n + 1) // blk) * blk
    srcg = jnp.pad(src, (0, e_pad - e)).reshape(_NW, k_chunks, _CHUNK)
    dstg = jnp.pad(dst, (0, e_pad - e)).reshape(_NW, k_chunks, _CHUNK)
    srcs = jnp.pad(src, (0, e_pad - e), constant_values=n).reshape(_NW, k_chunks, _CHUNK)
    dsts = jnp.pad(dst, (0, e_pad - e), constant_values=n).reshape(_NW, k_chunks, _CHUNK)
    zeros = jnp.zeros((acc_rows, DIM), jnp.float32)

    u = u_p.reshape(-1)[:acc_rows].reshape(acc_rows, 1)
    v = v_p.reshape(-1)[:acc_rows].reshape(acc_rows, 1)
    dinv = dinv_p.reshape(-1)[:acc_rows].reshape(acc_rows, 1)
    xp = jnp.pad(x, ((0, acc_rows - n), (0, 0)))

    def seg(table, gi, si):
        return _sc_seg_pass(table, gi, si, zeros, n, acc_rows, k_chunks)

    for _ in range(BLOCKS):
        out0, a_tab, b_tab, dh, y, yt = _stage_a(xp, W_lin, b_lin, W1, W2,
                                                 u, v, dinv)
        s1 = seg(a_tab, srcg, dsts)
        s2 = seg(b_tab, dstg, srcs)
        t1 = seg(y, dstg, srcs)
        t2 = seg(yt, srcg, dsts)

        pys, pty = _stage_b(t1, y, t2, yt, dinv)

        t3 = seg(pys, srcg, dsts)
        t4 = seg(pty, dstg, srcs)

        xp = _stage_c(out0, s1, s2, dh, t3, t4, pys, pty, u, v, dinv, b1, b2)
    return xp[:n]


# speedup vs baseline: 1.5027x; 1.4791x over previous
import functools
import math

import jax
import jax.numpy as jnp
from jax import lax
from jax.experimental import pallas as pl
from jax.experimental.pallas import tpu as pltpu
from jax.experimental.pallas import tpu_sc as plsc

N_NODES = 10000
DIM = 128
ALPHA_ITERS = 20
BLOCKS = 2

_ROWS = 632

_NC = 2
_NS = 16
_NW = _NC * _NS
_L = 16
_CHUNK = 128


def _sc_mesh():
    return plsc.VectorSubcoreMesh(
        core_axis_name="c", subcore_axis_name="s", num_cores=_NC, num_subcores=_NS)


def _sc_seg_pass(table, gidx, sidx, zeros, n, acc_rows, k_chunks):
    z_per_tile = acc_rows // _NS

    def body(table_h, gidx_h, sidx_h, zeros_h, out_h,
             gidx_v, sidx_v, rb0, acc_sh, semA):
        c = lax.axis_index("c")
        s = lax.axis_index("s")
        wid = c * _NS + s
        pltpu.sync_copy(gidx_h.at[wid], gidx_v)
        pltpu.sync_copy(sidx_h.at[wid], sidx_v)
        pltpu.sync_copy(zeros_h.at[pl.ds(s * z_per_tile, z_per_tile)],
                        acc_sh.at[pl.ds(s * z_per_tile, z_per_tile)])
        plsc.subcore_barrier()

        def chunk(j, carry):
            pltpu.async_copy(table_h.at[gidx_v.at[j]], rb0, semA).wait()
            pltpu.sync_copy(rb0, acc_sh.at[sidx_v.at[j]], add=True)
            return carry

        lax.fori_loop(0, k_chunks, chunk, 0, unroll=False)

        plsc.subcore_barrier()
        pltpu.sync_copy(
            acc_sh.at[pl.ds(s * z_per_tile, z_per_tile)],
            out_h.at[pl.ds(c * acc_rows + s * z_per_tile, z_per_tile)])

    return pl.kernel(
        body,
        out_type=jax.ShapeDtypeStruct((2 * acc_rows, DIM), jnp.float32),
        mesh=_sc_mesh(),
        scratch_types=[
            pltpu.VMEM((k_chunks, _CHUNK), jnp.int32),
            pltpu.VMEM((k_chunks, _CHUNK), jnp.int32),
            pltpu.VMEM((_CHUNK, DIM), jnp.float32),
            pltpu.VMEM_SHARED((acc_rows, DIM), jnp.float32),
            pltpu.SemaphoreType.DMA,
        ],
    )(table, gidx, sidx, zeros)


def _row_spec():
    return pl.BlockSpec((_ROWS, DIM), lambda i: (i, 0))


def _col_spec():
    return pl.BlockSpec((_ROWS, 1), lambda i: (i, 0))


def _stage_a_body(x_ref, wl_ref, bl_ref, w1_ref, w2_ref, u_ref, v_ref, d_ref,
                  out0_ref, a_ref, b_ref, dh_ref, y_ref, yt_ref):
    x = x_ref[...]
    out0_ref[...] = jnp.dot(x, wl_ref[...], preferred_element_type=jnp.float32) + bl_ref[...]
    h1 = jnp.dot(x, w1_ref[...], preferred_element_type=jnp.float32)
    y = jnp.dot(x, w2_ref[...], preferred_element_type=jnp.float32)
    a_ref[...] = u_ref[...] * h1
    b_ref[...] = v_ref[...] * h1
    dh_ref[...] = d_ref[...] * h1
    y_ref[...] = y
    yt_ref[...] = d_ref[...] * y


def _stage_a(x, W_lin, b_lin, W1, W2, u, v, dinv):
    n = x.shape[0]
    grid = (n // _ROWS,)
    full_spec = pl.BlockSpec((DIM, DIM), lambda i: (0, 0))
    bias_spec = pl.BlockSpec((1, DIM), lambda i: (0, 0))
    return pl.pallas_call(
        _stage_a_body,
        grid=grid,
        in_specs=[_row_spec(), full_spec, bias_spec, full_spec, full_spec,
                  _col_spec(), _col_spec(), _col_spec()],
        out_specs=[_row_spec()] * 6,
        out_shape=[jax.ShapeDtypeStruct((n, DIM), jnp.float32)] * 6,
    )(x, W_lin, b_lin.reshape(1, DIM), W1, W2, u, v, dinv)


def _stage_b_body(t1a_ref, t1b_ref, y_ref, t2a_ref, t2b_ref, yt_ref, d_ref,
                  pys_ref, pty_ref):
    d = d_ref[...]
    pys_ref[...] = d * d * (t1a_ref[...] + t1b_ref[...] + y_ref[...])
    pty_ref[...] = t2a_ref[...] + t2b_ref[...] + yt_ref[...]


def _stage_b(t1, y, t2, yt, dinv):
    n = y.shape[0]
    grid = (n // _ROWS,)
    lo = pl.BlockSpec((_ROWS, DIM), lambda i: (i, 0))
    hi = pl.BlockSpec((_ROWS, DIM), lambda i: (i + n // _ROWS, 0))
    return pl.pallas_call(
        _stage_b_body,
        grid=grid,
        in_specs=[lo, hi, _row_spec(), lo, hi, _row_spec(), _col_spec()],
        out_specs=[_row_spec()] * 2,
        out_shape=[jax.ShapeDtypeStruct((n, DIM), jnp.float32)] * 2,
    )(t1, t1, y, t2, t2, yt, dinv)


def _stage_c_body(out0_ref, s1a_ref, s1b_ref, s2a_ref, s2b_ref, dh_ref,
                  t3a_ref, t3b_ref, t4a_ref, t4b_ref, pys_ref, pty_ref,
                  u_ref, v_ref, d_ref, b1_ref, b2_ref, x_ref):
    out1 = (v_ref[...] * (s1a_ref[...] + s1b_ref[...])
            + u_ref[...] * (s2a_ref[...] + s2b_ref[...])
            + dh_ref[...] + b1_ref[...])
    l_in = t3a_ref[...] + t3b_ref[...] + pys_ref[...]
    l_out = d_ref[...] * (t4a_ref[...] + t4b_ref[...] + pty_ref[...])
    x_ref[...] = out0_ref[...] + out1 + 0.5 * (l_in + l_out) + b2_ref[...]


def _stage_c(out0, s1, s2, dh, t3, t4, pys, pty, u, v, dinv, b1, b2):
    n = out0.shape[0]
    grid = (n // _ROWS,)
    lo = pl.BlockSpec((_ROWS, DIM), lambda i: (i, 0))
    hi = pl.BlockSpec((_ROWS, DIM), lambda i: (i + n // _ROWS, 0))
    bias_spec = pl.BlockSpec((1, DIM), lambda i: (0, 0))
    return pl.pallas_call(
        _stage_c_body,
        grid=grid,
        in_specs=[_row_spec(), lo, hi, lo, hi, _row_spec(), lo, hi, lo, hi,
                  _row_spec(), _row_spec(), _col_spec(), _col_spec(),
                  _col_spec(), bias_spec, bias_spec],
        out_specs=_row_spec(),
        out_shape=jax.ShapeDtypeStruct((n, DIM), jnp.float32),
    )(out0, s1, s1, s2, s2, dh, t3, t3, t4, t4, pys, pty, u, v, dinv,
      b1.reshape(1, DIM), b2.reshape(1, DIM))


def _scalar_prep_body(pi_ref, d_ref, u_ref, v_ref):
    pis = jnp.sqrt(jnp.clip(pi_ref[...], 1e-12, None))
    u_ref[...] = 0.5 * d_ref[...] * pis
    v_ref[...] = 1.0 / pis


def _scalar_prep(pi_p, dinv_p):
    shp = pi_p.shape
    return pl.pallas_call(
        _scalar_prep_body,
        out_shape=[jax.ShapeDtypeStruct(shp, jnp.float32)] * 2,
    )(pi_p, dinv_p)


def _sc_power_iter(srcf, dstf, ca_in, cb_in, np_rows):
    rpt = np_rows // _NS
    nv = rpt // _L
    n_ch = srcf.shape[1]
    fpc = _CHUNK // _L

    def body(srcf_h, dstf_h, ca_h, cb_h, pi_h, dinv_h,
             src2, dst2, q_tab, vb, ones_b, zbuf, slc, dinv_l, pi_l, qs,
             ca, cb, sem0, sem1, acc_sh, q_sh):
        c = lax.axis_index("c")
        s = lax.axis_index("s")

        @pl.when(c == 0)
        def _run():
            base = s * rpt
            pltpu.sync_copy(srcf_h.at[s], src2)
            pltpu.sync_copy(dstf_h.at[s], dst2)
            pltpu.sync_copy(ca_h, ca)
            pltpu.sync_copy(cb_h, cb)
            zeros16 = jnp.zeros((_L,), jnp.float32)
            ones16 = jnp.ones((_L,), jnp.float32)
            for t in range(fpc):
                ones_b[pl.ds(t * _L, _L)] = ones16

            def zero16(j, _):
                zbuf[pl.ds(j * _L, _L)] = zeros16
                return _

            lax.fori_loop(0, nv, zero16, 0)
            pltpu.sync_copy(zbuf, acc_sh.at[pl.ds(base, rpt)])
            plsc.subcore_barrier()

            def hpair(i, _):
                d0 = pltpu.async_copy(ones_b, acc_sh.at[src2.at[2 * i]],
                                      sem0, add=True)
                d1 = pltpu.async_copy(ones_b, acc_sh.at[src2.at[2 * i + 1]],
                                      sem1, add=True)
                d0.wait()
                d1.wait()
                return _

            lax.fori_loop(0, n_ch // 2, hpair, 0)
            plsc.subcore_barrier()
            pltpu.sync_copy(acc_sh.at[pl.ds(base, rpt)], slc)
            inv_n = 1.0 / float(N_NODES)

            def dstep(j, _):
                di = 1.0 / (slc[pl.ds(j * _L, _L)] + 1.0)
                dinv_l[pl.ds(j * _L, _L)] = di
                qs[pl.ds(j * _L, _L)] = di * inv_n
                return _

            lax.fori_loop(0, nv, dstep, 0)
            pltpu.sync_copy(qs, q_sh.at[pl.ds(base, rpt)])
            pltpu.sync_copy(zbuf, acc_sh.at[pl.ds(base, rpt)])
            plsc.subcore_barrier()
            pltpu.sync_copy(q_sh, q_tab)

            def gath(k, b, sem):
                return pltpu.async_copy(q_sh.at[src2.at[k]], vb.at[b], sem)

            def one_iter(it, _):
                g0 = gath(0, 0, sem0)
                g0.wait()

                def pair(i, _c):
                    k0 = 2 * i
                    d0 = pltpu.async_copy(vb.at[0], acc_sh.at[dst2.at[k0]],
                                          sem0, add=True)
                    g1 = gath(k0 + 1, 1, sem1)
                    g1.wait()
                    d1 = pltpu.async_copy(vb.at[1], acc_sh.at[dst2.at[k0 + 1]],
                                          sem1, add=True)
                    d0.wait()

                    @pl.when(k0 + 2 < n_ch)
                    def _():
                        gath(k0 + 2, 0, sem0).wait()

                    d1.wait()
                    return _c

                lax.fori_loop(0, n_ch // 2, pair, 0)
                plsc.subcore_barrier()
                pltpu.sync_copy(acc_sh.at[pl.ds(base, rpt)], slc)
                A = ca[...]
                B = cb[...]

                def ustep(j, _c):
                    acc16 = slc[pl.ds(j * _L, _L)] + q_tab[pl.ds(base + j * _L, _L)]
                    pi16 = A * acc16 + B
                    pi_l[pl.ds(j * _L, _L)] = pi16
                    qs[pl.ds(j * _L, _L)] = dinv_l[pl.ds(j * _L, _L)] * pi16
                    return _c

                lax.fori_loop(0, nv, ustep, 0)
                pltpu.sync_copy(qs, q_sh.at[pl.ds(base, rpt)])
                pltpu.sync_copy(zbuf, acc_sh.at[pl.ds(base, rpt)])
                plsc.subcore_barrier()
                pltpu.sync_copy(q_sh, q_tab)
                return _

            lax.fori_loop(0, ALPHA_ITERS, one_iter, 0)
            pltpu.sync_copy(pi_l, pi_h.at[pl.ds(base, rpt)])
            pltpu.sync_copy(dinv_l, dinv_h.at[pl.ds(base, rpt)])

    return pl.kernel(
        body,
        out_type=(jax.ShapeDtypeStruct((np_rows,), jnp.float32),
                  jax.ShapeDtypeStruct((np_rows,), jnp.float32)),
        mesh=_sc_mesh(),
        scratch_types=[
            pltpu.VMEM((n_ch, _CHUNK), jnp.int32),
            pltpu.VMEM((n_ch, _CHUNK), jnp.int32),
            pltpu.VMEM((np_rows,), jnp.float32),
            pltpu.VMEM((2, _CHUNK), jnp.float32),
            pltpu.VMEM((_CHUNK,), jnp.float32),
            pltpu.VMEM((np_rows // _NS,), jnp.float32),
            pltpu.VMEM((np_rows // _NS,), jnp.float32),
            pltpu.VMEM((np_rows // _NS,), jnp.float32),
            pltpu.VMEM((np_rows // _NS,), jnp.float32),
            pltpu.VMEM((np_rows // _NS,), jnp.float32),
            pltpu.VMEM((_L,), jnp.float32),
            pltpu.VMEM((_L,), jnp.float32),
            pltpu.SemaphoreType.DMA,
            pltpu.SemaphoreType.DMA,
            pltpu.VMEM_SHARED((np_rows,), jnp.float32),
            pltpu.VMEM_SHARED((np_rows,), jnp.float32),
        ],
    )(srcf, dstf, ca_in, cb_in)


def kernel(x, edge_index, alpha, W_lin, b_lin, W1, b1, W2, b2):
    n = x.shape[0]
    src = edge_index[0]
    dst = edge_index[1]

    np_rows = -(-(n + 1) // (_NS * _L * 8)) * (_NS * _L * 8)
    ca_in = jnp.full((_L,), 1.0, jnp.float32) - alpha
    cb_in = jnp.full((_L,), 1.0 / n, jnp.float32) * alpha
    ept = src.shape[0] // _NS
    ept_pad = -(-ept // (2 * _CHUNK)) * (2 * _CHUNK)
    srcp = jnp.pad(src.reshape(_NS, ept), ((0, 0), (0, ept_pad - ept)),
                   constant_values=n).reshape(_NS, -1, _CHUNK)
    dstp = jnp.pad(dst.reshape(_NS, ept), ((0, 0), (0, ept_pad - ept)),
                   constant_values=n).reshape(_NS, -1, _CHUNK)
    pi_p, dinv_p = _sc_power_iter(srcp, dstp, ca_in, cb_in, np_rows)
    u_p, v_p = _scalar_prep(pi_p, dinv_p)

    e = src.shape[0]
    e_pad = -(-e // (_NW * _CHUNK)) * (_NW * _CHUNK)
    k_chunks = e_pad // (_NW * _CHUNK)
    blk = math.lcm(_NS * 8, _ROWS)
    acc_rows = -(-(n + 1) // blk) * blk
    srcg = jnp.pad(src, (0, e_pad - e)).reshape(_NW, k_chunks, _CHUNK)
    dstg = jnp.pad(dst, (0, e_pad - e)).reshape(_NW, k_chunks, _CHUNK)
    srcs = jnp.pad(src, (0, e_pad - e), constant_values=n).reshape(_NW, k_chunks, _CHUNK)
    dsts = jnp.pad(dst, (0, e_pad - e), constant_values=n).reshape(_NW, k_chunks, _CHUNK)
    zeros = jnp.zeros((acc_rows, DIM), jnp.float32)

    u = u_p.reshape(-1)[:acc_rows].reshape(acc_rows, 1)
    v = v_p.reshape(-1)[:acc_rows].reshape(acc_rows, 1)
    dinv = dinv_p.reshape(-1)[:acc_rows].reshape(acc_rows, 1)
    xp = jnp.pad(x, ((0, acc_rows - n), (0, 0)))

    def seg(table, gi, si):
        return _sc_seg_pass(table, gi, si, zeros, n, acc_rows, k_chunks)

    for _ in range(BLOCKS):
        out0, a_tab, b_tab, dh, y, yt = _stage_a(xp, W_lin, b_lin, W1, W2,
                                                 u, v, dinv)
        s1 = seg(a_tab, srcg, dsts)
        s2 = seg(b_tab, dstg, srcs)
        t1 = seg(y, dstg, srcs)
        t2 = seg(yt, srcg, dsts)

        pys, pty = _stage_b(t1, y, t2, yt, dinv)

        t3 = seg(pys, srcg, dsts)
        t4 = seg(pty, dstg, srcs)

        xp = _stage_c(out0, s1, s2, dh, t3, t4, pys, pty, u, v, dinv, b1, b2)
    return xp[:n]


# speedup vs baseline: 1.5037x; 1.0007x over previous
import math

import jax
import jax.numpy as jnp
from jax import lax
from jax.experimental import pallas as pl
from jax.experimental.pallas import tpu as pltpu
from jax.experimental.pallas import tpu_sc as plsc

N_NODES = 10000
DIM = 128
ALPHA_ITERS = 20
BLOCKS = 2

_ROWS = 632

_NC = 2
_NS = 16
_NW = _NC * _NS
_L = 16
_CHUNK = 128


def _sc_mesh():
    return plsc.VectorSubcoreMesh(
        core_axis_name="c", subcore_axis_name="s", num_cores=_NC, num_subcores=_NS)


def _sc_seg_pass(table, gidx, sidx, zeros, n, acc_rows, k_chunks):
    z_per_tile = acc_rows // _NS

    def body(table_h, gidx_h, sidx_h, zeros_h, out_h,
             gidx_v, sidx_v, rb0, acc_sh, semA):
        c = lax.axis_index("c")
        s = lax.axis_index("s")
        wid = c * _NS + s
        pltpu.sync_copy(gidx_h.at[wid], gidx_v)
        pltpu.sync_copy(sidx_h.at[wid], sidx_v)
        pltpu.sync_copy(zeros_h.at[pl.ds(s * z_per_tile, z_per_tile)],
                        acc_sh.at[pl.ds(s * z_per_tile, z_per_tile)])
        plsc.subcore_barrier()

        def chunk(j, carry):
            pltpu.async_copy(table_h.at[gidx_v.at[j]], rb0, semA).wait()
            pltpu.sync_copy(rb0, acc_sh.at[sidx_v.at[j]], add=True)
            return carry

        lax.fori_loop(0, k_chunks, chunk, 0, unroll=False)

        plsc.subcore_barrier()
        pltpu.sync_copy(
            acc_sh.at[pl.ds(s * z_per_tile, z_per_tile)],
            out_h.at[pl.ds(c * acc_rows + s * z_per_tile, z_per_tile)])

    return pl.kernel(
        body,
        out_type=jax.ShapeDtypeStruct((2 * acc_rows, DIM), jnp.float32),
        mesh=_sc_mesh(),
        scratch_types=[
            pltpu.VMEM((k_chunks, _CHUNK), jnp.int32),
            pltpu.VMEM((k_chunks, _CHUNK), jnp.int32),
            pltpu.VMEM((_CHUNK, DIM), jnp.float32),
            pltpu.VMEM_SHARED((acc_rows, DIM), jnp.float32),
            pltpu.SemaphoreType.DMA,
        ],
    )(table, gidx, sidx, zeros)


def _row_spec():
    return pl.BlockSpec((_ROWS, DIM), lambda i: (i, 0))


def _col_spec():
    return pl.BlockSpec((_ROWS, 1), lambda i: (i, 0))


def _stage_a_body(x_ref, wl_ref, bl_ref, w1_ref, w2_ref, u_ref, v_ref, d_ref,
                  out0_ref, a_ref, b_ref, dh_ref, y_ref, yt_ref):
    x = x_ref[...]
    out0_ref[...] = jnp.dot(x, wl_ref[...], preferred_element_type=jnp.float32) + bl_ref[...]
    h1 = jnp.dot(x, w1_ref[...], preferred_element_type=jnp.float32)
    y = jnp.dot(x, w2_ref[...], preferred_element_type=jnp.float32)
    a_ref[...] = u_ref[...] * h1
    b_ref[...] = v_ref[...] * h1
    dh_ref[...] = d_ref[...] * h1
    y_ref[...] = y
    yt_ref[...] = d_ref[...] * y


def _stage_a(x, W_lin, b_lin, W1, W2, u, v, dinv):
    n = x.shape[0]
    grid = (n // _ROWS,)
    full_spec = pl.BlockSpec((DIM, DIM), lambda i: (0, 0))
    bias_spec = pl.BlockSpec((1, DIM), lambda i: (0, 0))
    return pl.pallas_call(
        _stage_a_body,
        grid=grid,
        in_specs=[_row_spec(), full_spec, bias_spec, full_spec, full_spec,
                  _col_spec(), _col_spec(), _col_spec()],
        out_specs=[_row_spec()] * 6,
        out_shape=[jax.ShapeDtypeStruct((n, DIM), jnp.float32)] * 6,
    )(x, W_lin, b_lin.reshape(1, DIM), W1, W2, u, v, dinv)


def _stage_b_body(t1a_ref, t1b_ref, y_ref, t2a_ref, t2b_ref, yt_ref, d_ref,
                  pys_ref, pty_ref):
    d = d_ref[...]
    pys_ref[...] = d * d * (t1a_ref[...] + t1b_ref[...] + y_ref[...])
    pty_ref[...] = t2a_ref[...] + t2b_ref[...] + yt_ref[...]


def _stage_b(t1, y, t2, yt, dinv):
    n = y.shape[0]
    grid = (n // _ROWS,)
    lo = pl.BlockSpec((_ROWS, DIM), lambda i: (i, 0))
    hi = pl.BlockSpec((_ROWS, DIM), lambda i: (i + n // _ROWS, 0))
    return pl.pallas_call(
        _stage_b_body,
        grid=grid,
        in_specs=[lo, hi, _row_spec(), lo, hi, _row_spec(), _col_spec()],
        out_specs=[_row_spec()] * 2,
        out_shape=[jax.ShapeDtypeStruct((n, DIM), jnp.float32)] * 2,
    )(t1, t1, y, t2, t2, yt, dinv)


def _stage_c_body(out0_ref, s1a_ref, s1b_ref, s2a_ref, s2b_ref, dh_ref,
                  t3a_ref, t3b_ref, t4a_ref, t4b_ref, pys_ref, pty_ref,
                  u_ref, v_ref, d_ref, b1_ref, b2_ref, x_ref):
    out1 = (v_ref[...] * (s1a_ref[...] + s1b_ref[...])
            + u_ref[...] * (s2a_ref[...] + s2b_ref[...])
            + dh_ref[...] + b1_ref[...])
    l_in = t3a_ref[...] + t3b_ref[...] + pys_ref[...]
    l_out = d_ref[...] * (t4a_ref[...] + t4b_ref[...] + pty_ref[...])
    x_ref[...] = out0_ref[...] + out1 + 0.5 * (l_in + l_out) + b2_ref[...]


def _stage_c(out0, s1, s2, dh, t3, t4, pys, pty, u, v, dinv, b1, b2):
    n = out0.shape[0]
    grid = (n // _ROWS,)
    lo = pl.BlockSpec((_ROWS, DIM), lambda i: (i, 0))
    hi = pl.BlockSpec((_ROWS, DIM), lambda i: (i + n // _ROWS, 0))
    bias_spec = pl.BlockSpec((1, DIM), lambda i: (0, 0))
    return pl.pallas_call(
        _stage_c_body,
        grid=grid,
        in_specs=[_row_spec(), lo, hi, lo, hi, _row_spec(), lo, hi, lo, hi,
                  _row_spec(), _row_spec(), _col_spec(), _col_spec(),
                  _col_spec(), bias_spec, bias_spec],
        out_specs=_row_spec(),
        out_shape=jax.ShapeDtypeStruct((n, DIM), jnp.float32),
    )(out0, s1, s1, s2, s2, dh, t3, t3, t4, t4, pys, pty, u, v, dinv,
      b1.reshape(1, DIM), b2.reshape(1, DIM))


def _scalar_prep_body(pi_ref, d_ref, u_ref, v_ref):
    pis = jnp.sqrt(jnp.clip(pi_ref[...], 1e-12, None))
    u_ref[...] = 0.5 * d_ref[...] * pis
    v_ref[...] = 1.0 / pis


def _scalar_prep(pi_p, dinv_p):
    shp = pi_p.shape
    return pl.pallas_call(
        _scalar_prep_body,
        out_shape=[jax.ShapeDtypeStruct(shp, jnp.float32)] * 2,
    )(pi_p, dinv_p)


def _sc_power_iter(srcf, dstf, ca_in, cb_in, np_rows):
    rpt = np_rows // _NS
    nv = rpt // _L
    n_ch = srcf.shape[1]
    fpc = _CHUNK // _L

    def body(srcf_h, dstf_h, ca_h, cb_h, pi_h, dinv_h,
             src2, dst2, q_tab, vb, ones_b, zbuf, slc, dinv_l, pi_l, qs,
             ca, cb, sem0, sem1, acc_sh, q_sh):
        c = lax.axis_index("c")
        s = lax.axis_index("s")

        @pl.when(c == 0)
        def _run():
            base = s * rpt
            pltpu.sync_copy(srcf_h.at[s], src2)
            pltpu.sync_copy(dstf_h.at[s], dst2)
            pltpu.sync_copy(ca_h, ca)
            pltpu.sync_copy(cb_h, cb)
            zeros16 = jnp.zeros((_L,), jnp.float32)
            ones16 = jnp.ones((_L,), jnp.float32)
            for t in range(fpc):
                ones_b[pl.ds(t * _L, _L)] = ones16

            def zero16(j, _):
                zbuf[pl.ds(j * _L, _L)] = zeros16
                return _

            lax.fori_loop(0, nv, zero16, 0)
            pltpu.sync_copy(zbuf, acc_sh.at[pl.ds(base, rpt)])
            plsc.subcore_barrier()

            def hpair(i, _):
                d0 = pltpu.async_copy(ones_b, acc_sh.at[src2.at[2 * i]],
                                      sem0, add=True)
                d1 = pltpu.async_copy(ones_b, acc_sh.at[src2.at[2 * i + 1]],
                                      sem1, add=True)
                d0.wait()
                d1.wait()
                return _

            lax.fori_loop(0, n_ch // 2, hpair, 0)
            plsc.subcore_barrier()
            pltpu.sync_copy(acc_sh.at[pl.ds(base, rpt)], slc)
            inv_n = 1.0 / float(N_NODES)

            def dstep(j, _):
                di = 1.0 / (slc[pl.ds(j * _L, _L)] + 1.0)
                dinv_l[pl.ds(j * _L, _L)] = di
                qs[pl.ds(j * _L, _L)] = di * inv_n
                return _

            lax.fori_loop(0, nv, dstep, 0)
            pltpu.sync_copy(qs, q_sh.at[pl.ds(base, rpt)])
            pltpu.sync_copy(zbuf, acc_sh.at[pl.ds(base, rpt)])
            plsc.subcore_barrier()
            pltpu.sync_copy(q_sh, q_tab)

            def gath(k, b, sem):
                return pltpu.async_copy(q_sh.at[src2.at[k]], vb.at[b], sem)

            def one_iter(it, _):
                g0 = gath(0, 0, sem0)
                g0.wait()

                def pair(i, _c):
                    k0 = 2 * i
                    d0 = pltpu.async_copy(vb.at[0], acc_sh.at[dst2.at[k0]],
                                          sem0, add=True)
                    g1 = gath(k0 + 1, 1, sem1)
                    g1.wait()
                    d1 = pltpu.async_copy(vb.at[1], acc_sh.at[dst2.at[k0 + 1]],
                                          sem1, add=True)
                    d0.wait()

                    @pl.when(k0 + 2 < n_ch)
                    def _():
                        gath(k0 + 2, 0, sem0).wait()

                    d1.wait()
                    return _c

                lax.fori_loop(0, n_ch // 2, pair, 0)
                plsc.subcore_barrier()
                pltpu.sync_copy(acc_sh.at[pl.ds(base, rpt)], slc)
                A = ca[...]
                B = cb[...]

                def ustep(j, _c):
                    acc16 = slc[pl.ds(j * _L, _L)] + q_tab[pl.ds(base + j * _L, _L)]
                    pi16 = A * acc16 + B
                    pi_l[pl.ds(j * _L, _L)] = pi16
                    qs[pl.ds(j * _L, _L)] = dinv_l[pl.ds(j * _L, _L)] * pi16
                    return _c

                lax.fori_loop(0, nv, ustep, 0)
                pltpu.sync_copy(qs, q_sh.at[pl.ds(base, rpt)])
                pltpu.sync_copy(zbuf, acc_sh.at[pl.ds(base, rpt)])
                plsc.subcore_barrier()
                pltpu.sync_copy(q_sh, q_tab)
                return _

            lax.fori_loop(0, ALPHA_ITERS, one_iter, 0)
            pltpu.sync_copy(pi_l, pi_h.at[pl.ds(base, rpt)])
            pltpu.sync_copy(dinv_l, dinv_h.at[pl.ds(base, rpt)])

    return pl.kernel(
        body,
        out_type=(jax.ShapeDtypeStruct((np_rows,), jnp.float32),
                  jax.ShapeDtypeStruct((np_rows,), jnp.float32)),
        mesh=_sc_mesh(),
        scratch_types=[
            pltpu.VMEM((n_ch, _CHUNK), jnp.int32),
            pltpu.VMEM((n_ch, _CHUNK), jnp.int32),
            pltpu.VMEM((np_rows,), jnp.float32),
            pltpu.VMEM((2, _CHUNK), jnp.float32),
            pltpu.VMEM((_CHUNK,), jnp.float32),
            pltpu.VMEM((np_rows // _NS,), jnp.float32),
            pltpu.VMEM((np_rows // _NS,), jnp.float32),
            pltpu.VMEM((np_rows // _NS,), jnp.float32),
            pltpu.VMEM((np_rows // _NS,), jnp.float32),
            pltpu.VMEM((np_rows // _NS,), jnp.float32),
            pltpu.VMEM((_L,), jnp.float32),
            pltpu.VMEM((_L,), jnp.float32),
            pltpu.SemaphoreType.DMA,
            pltpu.SemaphoreType.DMA,
            pltpu.VMEM_SHARED((np_rows,), jnp.float32),
            pltpu.VMEM_SHARED((np_rows,), jnp.float32),
        ],
    )(srcf, dstf, ca_in, cb_in)


def kernel(x, edge_index, alpha, W_lin, b_lin, W1, b1, W2, b2):
    n = x.shape[0]
    src = edge_index[0]
    dst = edge_index[1]

    np_rows = -(-(n + 1) // (_NS * _L * 8)) * (_NS * _L * 8)
    ca_in = jnp.full((_L,), 1.0, jnp.float32) - alpha
    cb_in = jnp.full((_L,), 1.0 / n, jnp.float32) * alpha
    ept = src.shape[0] // _NS
    ept_pad = -(-ept // (2 * _CHUNK)) * (2 * _CHUNK)
    srcp = jnp.pad(src.reshape(_NS, ept), ((0, 0), (0, ept_pad - ept)),
                   constant_values=n).reshape(_NS, -1, _CHUNK)
    dstp = jnp.pad(dst.reshape(_NS, ept), ((0, 0), (0, ept_pad - ept)),
                   constant_values=n).reshape(_NS, -1, _CHUNK)
    pi_p, dinv_p = _sc_power_iter(srcp, dstp, ca_in, cb_in, np_rows)
    u_p, v_p = _scalar_prep(pi_p, dinv_p)

    e = src.shape[0]
    e_pad = -(-e // (_NW * _CHUNK)) * (_NW * _CHUNK)
    k_chunks = e_pad // (_NW * _CHUNK)
    blk = math.lcm(_NS * 8, _ROWS)
    acc_rows = -(-(n + 1) // blk) * blk
    srcg = jnp.pad(src, (0, e_pad - e)).reshape(_NW, k_chunks, _CHUNK)
    dstg = jnp.pad(dst, (0, e_pad - e)).reshape(_NW, k_chunks, _CHUNK)
    srcs = jnp.pad(src, (0, e_pad - e), constant_values=n).reshape(_NW, k_chunks, _CHUNK)
    dsts = jnp.pad(dst, (0, e_pad - e), constant_values=n).reshape(_NW, k_chunks, _CHUNK)
    zeros = jnp.zeros((acc_rows, DIM), jnp.float32)

    u = u_p.reshape(-1)[:acc_rows].reshape(acc_rows, 1)
    v = v_p.reshape(-1)[:acc_rows].reshape(acc_rows, 1)
    dinv = dinv_p.reshape(-1)[:acc_rows].reshape(acc_rows, 1)
    xp = jnp.pad(x, ((0, acc_rows - n), (0, 0)))

    def seg(table, gi, si):
        return _sc_seg_pass(table, gi, si, zeros, n, acc_rows, k_chunks)

    for _ in range(BLOCKS):
        out0, a_tab, b_tab, dh, y, yt = _stage_a(xp, W_lin, b_lin, W1, W2,
                                                 u, v, dinv)
        s1 = seg(a_tab, srcg, dsts)
        s2 = seg(b_tab, dstg, srcs)
        t1 = seg(y, dstg, srcs)
        t2 = seg(yt, srcg, dsts)

        pys, pty = _stage_b(t1, y, t2, yt, dinv)

        t3 = seg(pys, srcg, dsts)
        t4 = seg(pty, dstg, srcs)

        xp = _stage_c(out0, s1, s2, dh, t3, t4, pys, pty, u, v, dinv, b1, b2)
    return xp[:n]
